# baseline (speedup 1.0000x reference)
import functools

import jax
import jax.numpy as jnp
from jax import lax
from jax.experimental import pallas as pl
from jax.experimental.pallas import tpu as pltpu
from jax.experimental.pallas import tpu_sc as plsc

E, D, F = 8, 2048, 5632
T = 256
FB = 512
NF = F // FB
N_TOK = 4096
P = N_TOK + E * T
SEG = 2048
TPS = SEG // T
NSEG = E + 1


_SC_INFO = plsc.get_sparse_core_info()
_SC_NC = _SC_INFO.num_cores
_SC_NW = _SC_NC * _SC_INFO.num_subcores


def _sc_gather_rows(table, idx, n_out, chunk):
    rows_shape = table.shape[1:]
    per_w = n_out // _SC_NW
    n_chunks = per_w // chunk
    assert per_w * _SC_NW == n_out and n_chunks * chunk == per_w
    mesh = plsc.VectorSubcoreMesh(core_axis_name="c", subcore_axis_name="s")

    @functools.partial(
        pl.kernel, mesh=mesh,
        out_type=jax.ShapeDtypeStruct((n_out,) + rows_shape, table.dtype),
        scratch_types=[
            pltpu.VMEM((per_w,), jnp.int32),
            pltpu.VMEM((2, chunk) + rows_shape, table.dtype),
            pltpu.SemaphoreType.DMA,
            pltpu.SemaphoreType.DMA,
            pltpu.SemaphoreType.DMA,
            pltpu.SemaphoreType.DMA,
        ],
    )
    def gk(table_hbm, idx_hbm, out_hbm, idx_v, rows_v,
           g_sem0, g_sem1, s_sem0, s_sem1):
        g_sems = (g_sem0, g_sem1)
        s_sems = (s_sem0, s_sem1)
        wid = lax.axis_index("s") * _SC_NC + lax.axis_index("c")
        base = wid * per_w
        pltpu.sync_copy(idx_hbm.at[pl.ds(base, per_w)], idx_v)

        def gather(c):
            return pltpu.make_async_copy(
                table_hbm.at[idx_v.at[pl.ds(c * chunk, chunk)]],
                rows_v.at[c % 2], g_sems[c % 2])

        def store(c):
            return pltpu.make_async_copy(
                rows_v.at[c % 2],
                out_hbm.at[pl.ds(base + c * chunk, chunk)], s_sems[c % 2])

        gather(0).start()
        for c in range(n_chunks):
            gather(c).wait()
            if c >= 1:
                store(c - 1).wait()
            if c + 1 < n_chunks:
                gather(c + 1).start()
            store(c).start()
        store(n_chunks - 1).wait()

    return gk(table, idx)


def _grouped_ffn_body(se_ref, snt_ref, soff_ref, w1_ref, w3_ref, w2_ref,
                      x_hbm, out_hbm, x_seg, acc_ref, wb1, wb3, wb2,
                      ld_sem, st_sem):
    s = pl.program_id(0)
    f = pl.program_id(1)

    off = soff_ref[s]
    nt = snt_ref[s]

    @pl.when(nt > 0)
    def _work():
        @pl.when(f == 0)
        def _load_seg():
            def stage(k, carry):
                cp = pltpu.make_async_copy(
                    x_hbm.at[pl.ds(pl.multiple_of(off + k * T, T), T), :],
                    x_seg.at[pl.ds(pl.multiple_of(k * T, T), T), :],
                    ld_sem)
                cp.start()
                cp.wait()
                return carry
            lax.fori_loop(0, nt, stage, 0)

        wb1[...] = w1_ref[0].astype(jnp.bfloat16)
        wb3[...] = w3_ref[0].astype(jnp.bfloat16)
        wb2[...] = w2_ref[0].astype(jnp.bfloat16)

        def tile_body(k, carry):
            rows = x_seg[pl.ds(pl.multiple_of(k * T, T), T), :]
            g = jnp.dot(rows, wb1[...], preferred_element_type=jnp.float32)
            u = jnp.dot(rows, wb3[...], preferred_element_type=jnp.float32)
            h = (g * jax.nn.sigmoid(g)) * u
            contrib = jnp.dot(h.astype(jnp.bfloat16), wb2[...],
                              preferred_element_type=jnp.float32)
            sl = pl.ds(pl.multiple_of(k * T, T), T)

            @pl.when(f == 0)
            def _init():
                acc_ref[sl, :] = contrib

            @pl.when(f > 0)
            def _accum():
                acc_ref[sl, :] = acc_ref[sl, :] + contrib

            return carry

        lax.fori_loop(0, nt, tile_body, 0)

        @pl.when(f == NF - 1)
        def _flush():
            def flush_tile(k, carry):
                cp = pltpu.make_async_copy(
                    acc_ref.at[pl.ds(pl.multiple_of(k * T, T), T), :],
                    out_hbm.at[pl.ds(pl.multiple_of(off + k * T, T), T), :],
                    st_sem)
                cp.start()
                cp.wait()
                return carry
            lax.fori_loop(0, nt, flush_tile, 0)


def _grouped_ffn(x_sorted, seg_expert, seg_nt, seg_off, w1, w3, w2):
    def wmap_in(s, f, se, snt, soff):
        return (se[s], 0, jnp.where(snt[s] > 0, f, 0))

    def wmap_out(s, f, se, snt, soff):
        return (se[s], jnp.where(snt[s] > 0, f, 0), 0)

    grid_spec = pltpu.PrefetchScalarGridSpec(
        num_scalar_prefetch=3,
        grid=(NSEG, NF),
        in_specs=[
            pl.BlockSpec((1, D, FB), wmap_in),
            pl.BlockSpec((1, D, FB), wmap_in),
            pl.BlockSpec((1, FB, D), wmap_out),
            pl.BlockSpec(memory_space=pl.ANY),
        ],
        out_specs=pl.BlockSpec(memory_space=pl.ANY),
        scratch_shapes=[
            pltpu.VMEM((SEG, D), jnp.bfloat16),
            pltpu.VMEM((SEG, D), jnp.float32),
            pltpu.VMEM((D, FB), jnp.bfloat16),
            pltpu.VMEM((D, FB), jnp.bfloat16),
            pltpu.VMEM((FB, D), jnp.bfloat16),
            pltpu.SemaphoreType.DMA,
            pltpu.SemaphoreType.DMA,
        ],
    )
    return pl.pallas_call(
        _grouped_ffn_body,
        grid_spec=grid_spec,
        out_shape=jax.ShapeDtypeStruct((P, D), jnp.float32),
    )(seg_expert, seg_nt, seg_off, w1, w3, w2, x_sorted)


def kernel(x, expert_idx, w1, w3, w2):
    b, s, d = x.shape
    x_flat = x.reshape(-1, d)
    idx = expert_idx.reshape(-1).astype(jnp.int32)
    n = idx.shape[0]

    order = jnp.argsort(idx)
    sorted_e = jnp.take(idx, order)
    counts = jnp.sum(jax.nn.one_hot(idx, E, dtype=jnp.int32), axis=0)
    padded = ((counts + T - 1) // T) * T
    offs = jnp.concatenate([jnp.zeros((1,), jnp.int32),
                            jnp.cumsum(padded)[:-1].astype(jnp.int32)])
    ntiles = (padded // T).astype(jnp.int32)
    cum = jnp.concatenate([jnp.zeros((1,), jnp.int32),
                           jnp.cumsum(counts)[:-1].astype(jnp.int32)])
    dst = jnp.take(offs, sorted_e) + (jnp.arange(n, dtype=jnp.int32)
                                      - jnp.take(cum, sorted_e))
    src_of_pos = jnp.zeros((P,), jnp.int32).at[dst].set(order.astype(jnp.int32))
    pos_of_token = jnp.zeros((n,), jnp.int32).at[order].set(dst)

    segs_e = (ntiles + TPS - 1) // TPS
    seg_start = jnp.concatenate([jnp.zeros((1,), jnp.int32),
                                 jnp.cumsum(segs_e)[:-1].astype(jnp.int32)])
    s_ids = jnp.arange(NSEG, dtype=jnp.int32)
    seg_expert = jnp.clip(
        jnp.sum((seg_start[None, :] <= s_ids[:, None]).astype(jnp.int32),
                axis=1) - 1, 0, E - 1).astype(jnp.int32)
    k_of_seg = s_ids - jnp.take(seg_start, seg_expert)
    seg_off = (jnp.take(offs, seg_expert) + k_of_seg * SEG).astype(jnp.int32)
    seg_nt = jnp.clip(jnp.take(ntiles, seg_expert) - k_of_seg * TPS,
                      0, TPS).astype(jnp.int32)

    x_bf = x_flat.astype(jnp.bfloat16)
    x_i32 = lax.bitcast_convert_type(x_bf.reshape(n, d // 2, 2), jnp.int32)
    xs_i32 = _sc_gather_rows(x_i32, src_of_pos, P, 32)
    x_sorted = lax.bitcast_convert_type(xs_i32, jnp.bfloat16).reshape(P, d)

    y_sorted = _grouped_ffn(x_sorted, seg_expert, seg_nt, seg_off, w1, w3, w2)

    out = _sc_gather_rows(y_sorted, pos_of_token, n, 16)
    return out.reshape(b, s, d)


# speedup vs baseline: 1.4043x; 1.4043x over previous
import functools

import jax
import jax.numpy as jnp
from jax import lax
from jax.experimental import pallas as pl
from jax.experimental.pallas import tpu as pltpu
from jax.experimental.pallas import tpu_sc as plsc

E, D, F = 8, 2048, 5632
T = 256
FB = 512
NF = F // FB
N_TOK = 4096
P = N_TOK + E * T
SEG = 2048
TPS = SEG // T
NSEG = E + 1


_SC_INFO = plsc.get_sparse_core_info()
_SC_NC = _SC_INFO.num_cores
_SC_NW = _SC_NC * _SC_INFO.num_subcores


def _sc_gather_rows(table, idx, n_out, chunk):
    rows_shape = table.shape[1:]
    per_w = n_out // _SC_NW
    n_chunks = per_w // chunk
    assert per_w * _SC_NW == n_out and n_chunks * chunk == per_w
    mesh = plsc.VectorSubcoreMesh(core_axis_name="c", subcore_axis_name="s")

    @functools.partial(
        pl.kernel, mesh=mesh,
        out_type=jax.ShapeDtypeStruct((n_out,) + rows_shape, table.dtype),
        scratch_types=[
            pltpu.VMEM((per_w,), jnp.int32),
            pltpu.VMEM((2, chunk) + rows_shape, table.dtype),
            pltpu.SemaphoreType.DMA,
            pltpu.SemaphoreType.DMA,
            pltpu.SemaphoreType.DMA,
            pltpu.SemaphoreType.DMA,
        ],
    )
    def gk(table_hbm, idx_hbm, out_hbm, idx_v, rows_v,
           g_sem0, g_sem1, s_sem0, s_sem1):
        g_sems = (g_sem0, g_sem1)
        s_sems = (s_sem0, s_sem1)
        wid = lax.axis_index("s") * _SC_NC + lax.axis_index("c")
        base = wid * per_w
        pltpu.sync_copy(idx_hbm.at[pl.ds(base, per_w)], idx_v)

        def gather(c):
            return pltpu.make_async_copy(
                table_hbm.at[idx_v.at[pl.ds(c * chunk, chunk)]],
                rows_v.at[c % 2], g_sems[c % 2])

        def store(c):
            return pltpu.make_async_copy(
                rows_v.at[c % 2],
                out_hbm.at[pl.ds(base + c * chunk, chunk)], s_sems[c % 2])

        gather(0).start()
        for c in range(n_chunks):
            gather(c).wait()
            if c >= 1:
                store(c - 1).wait()
            if c + 1 < n_chunks:
                gather(c + 1).start()
            store(c).start()
        store(n_chunks - 1).wait()

    return gk(table, idx)


def _grouped_ffn_body(se_ref, snt_ref, soff_ref, w1_ref, w3_ref, w2_ref,
                      x_hbm, out_hbm, x_seg, acc_ref, wb1, wb3, wb2,
                      ld_sem, st_sem):
    s = pl.program_id(0)
    f = pl.program_id(1)

    off = soff_ref[s]
    nt = snt_ref[s]

    @pl.when(nt > 0)
    def _work():
        @pl.when(f == 0)
        def _load_seg():
            def stage(k, carry):
                cp = pltpu.make_async_copy(
                    x_hbm.at[pl.ds(pl.multiple_of(off + k * T, T), T), :],
                    x_seg.at[pl.ds(pl.multiple_of(k * T, T), T), :],
                    ld_sem)
                cp.start()
                cp.wait()
                return carry
            lax.fori_loop(0, nt, stage, 0)

        wb1[...] = w1_ref[0].astype(jnp.bfloat16)
        wb3[...] = w3_ref[0].astype(jnp.bfloat16)
        wb2[...] = w2_ref[0].astype(jnp.bfloat16)

        def tile_body(k, carry):
            rows = x_seg[pl.ds(pl.multiple_of(k * T, T), T), :]
            g = jnp.dot(rows, wb1[...], preferred_element_type=jnp.float32)
            u = jnp.dot(rows, wb3[...], preferred_element_type=jnp.float32)
            h = (g * jax.nn.sigmoid(g)) * u
            contrib = jnp.dot(h.astype(jnp.bfloat16), wb2[...],
                              preferred_element_type=jnp.float32)
            sl = pl.ds(pl.multiple_of(k * T, T), T)

            @pl.when(f == 0)
            def _init():
                acc_ref[sl, :] = contrib

            @pl.when(f > 0)
            def _accum():
                acc_ref[sl, :] = acc_ref[sl, :] + contrib

            return carry

        lax.fori_loop(0, nt, tile_body, 0)

        @pl.when(f == NF - 1)
        def _flush():
            def flush_tile(k, carry):
                cp = pltpu.make_async_copy(
                    acc_ref.at[pl.ds(pl.multiple_of(k * T, T), T), :],
                    out_hbm.at[pl.ds(pl.multiple_of(off + k * T, T), T), :],
                    st_sem)
                cp.start()
                cp.wait()
                return carry
            lax.fori_loop(0, nt, flush_tile, 0)


def _grouped_ffn(x_sorted, seg_expert, seg_nt, seg_off, w1, w3, w2):
    def wmap_in(s, f, se, snt, soff):
        return (se[s], 0, jnp.where(snt[s] > 0, f, 0))

    def wmap_out(s, f, se, snt, soff):
        return (se[s], jnp.where(snt[s] > 0, f, 0), 0)

    grid_spec = pltpu.PrefetchScalarGridSpec(
        num_scalar_prefetch=3,
        grid=(NSEG, NF),
        in_specs=[
            pl.BlockSpec((1, D, FB), wmap_in),
            pl.BlockSpec((1, D, FB), wmap_in),
            pl.BlockSpec((1, FB, D), wmap_out),
            pl.BlockSpec(memory_space=pl.ANY),
        ],
        out_specs=pl.BlockSpec(memory_space=pl.ANY),
        scratch_shapes=[
            pltpu.VMEM((SEG, D), jnp.bfloat16),
            pltpu.VMEM((SEG, D), jnp.float32),
            pltpu.VMEM((D, FB), jnp.bfloat16),
            pltpu.VMEM((D, FB), jnp.bfloat16),
            pltpu.VMEM((FB, D), jnp.bfloat16),
            pltpu.SemaphoreType.DMA,
            pltpu.SemaphoreType.DMA,
        ],
    )
    return pl.pallas_call(
        _grouped_ffn_body,
        grid_spec=grid_spec,
        out_shape=jax.ShapeDtypeStruct((P, D), jnp.float32),
    )(seg_expert, seg_nt, seg_off, w1, w3, w2, x_sorted)


def kernel(x, expert_idx, w1, w3, w2):
    b, s, d = x.shape
    x_flat = x.reshape(-1, d)
    idx = expert_idx.reshape(-1).astype(jnp.int32)
    n = idx.shape[0]

    order = jnp.argsort(idx)
    sorted_e = jnp.take(idx, order)
    counts = jnp.sum(jax.nn.one_hot(idx, E, dtype=jnp.int32), axis=0)
    padded = ((counts + T - 1) // T) * T
    offs = jnp.concatenate([jnp.zeros((1,), jnp.int32),
                            jnp.cumsum(padded)[:-1].astype(jnp.int32)])
    ntiles = (padded // T).astype(jnp.int32)
    cum = jnp.concatenate([jnp.zeros((1,), jnp.int32),
                           jnp.cumsum(counts)[:-1].astype(jnp.int32)])
    dst = jnp.take(offs, sorted_e) + (jnp.arange(n, dtype=jnp.int32)
                                      - jnp.take(cum, sorted_e))
    src_of_pos = jnp.zeros((P,), jnp.int32).at[dst].set(order.astype(jnp.int32))
    pos_of_token = jnp.zeros((n,), jnp.int32).at[order].set(dst)

    segs_e = (ntiles + TPS - 1) // TPS
    seg_start = jnp.concatenate([jnp.zeros((1,), jnp.int32),
                                 jnp.cumsum(segs_e)[:-1].astype(jnp.int32)])
    s_ids = jnp.arange(NSEG, dtype=jnp.int32)
    seg_expert = jnp.clip(
        jnp.sum((seg_start[None, :] <= s_ids[:, None]).astype(jnp.int32),
                axis=1) - 1, 0, E - 1).astype(jnp.int32)
    k_of_seg = s_ids - jnp.take(seg_start, seg_expert)
    seg_off = (jnp.take(offs, seg_expert) + k_of_seg * SEG).astype(jnp.int32)
    seg_nt = jnp.clip(jnp.take(ntiles, seg_expert) - k_of_seg * TPS,
                      0, TPS).astype(jnp.int32)

    x_sorted = _sc_gather_rows(x_flat, src_of_pos, P, 16).astype(jnp.bfloat16)

    y_sorted = _grouped_ffn(x_sorted, seg_expert, seg_nt, seg_off, w1, w3, w2)

    out = _sc_gather_rows(y_sorted, pos_of_token, n, 16)
    return out.reshape(b, s, d)


# speedup vs baseline: 1.4226x; 1.0130x over previous
import functools

import jax
import jax.numpy as jnp
from jax import lax
from jax.experimental import pallas as pl
from jax.experimental.pallas import tpu as pltpu
from jax.experimental.pallas import tpu_sc as plsc

E, D, F = 8, 2048, 5632
T = 256
FB = 512
NF = F // FB
N_TOK = 4096
P = N_TOK + E * T
SEG = 2048
TPS = SEG // T
NSEG = E + 1


_SC_INFO = plsc.get_sparse_core_info()
_SC_NC = _SC_INFO.num_cores
_SC_NW = _SC_NC * _SC_INFO.num_subcores


def _sc_gather_rows(table, idx, n_out, chunk):
    rows_shape = table.shape[1:]
    per_w = n_out // _SC_NW
    n_chunks = per_w // chunk
    assert per_w * _SC_NW == n_out and n_chunks * chunk == per_w
    mesh = plsc.VectorSubcoreMesh(core_axis_name="c", subcore_axis_name="s")

    @functools.partial(
        pl.kernel, mesh=mesh,
        out_type=jax.ShapeDtypeStruct((n_out,) + rows_shape, table.dtype),
        scratch_types=[
            pltpu.VMEM((per_w,), jnp.int32),
            pltpu.VMEM((2, chunk) + rows_shape, table.dtype),
            pltpu.SemaphoreType.DMA,
            pltpu.SemaphoreType.DMA,
            pltpu.SemaphoreType.DMA,
            pltpu.SemaphoreType.DMA,
        ],
    )
    def gk(table_hbm, idx_hbm, out_hbm, idx_v, rows_v,
           g_sem0, g_sem1, s_sem0, s_sem1):
        g_sems = (g_sem0, g_sem1)
        s_sems = (s_sem0, s_sem1)
        wid = lax.axis_index("s") * _SC_NC + lax.axis_index("c")
        base = wid * per_w
        pltpu.sync_copy(idx_hbm.at[pl.ds(base, per_w)], idx_v)

        def gather(c):
            return pltpu.make_async_copy(
                table_hbm.at[idx_v.at[pl.ds(c * chunk, chunk)]],
                rows_v.at[c % 2], g_sems[c % 2])

        def store(c):
            return pltpu.make_async_copy(
                rows_v.at[c % 2],
                out_hbm.at[pl.ds(base + c * chunk, chunk)], s_sems[c % 2])

        gather(0).start()
        for c in range(n_chunks):
            gather(c).wait()
            if c >= 1:
                store(c - 1).wait()
            if c + 1 < n_chunks:
                gather(c + 1).start()
            store(c).start()
        store(n_chunks - 1).wait()

    return gk(table, idx)


def _grouped_ffn_body(se_ref, snt_ref, soff_ref, w1_ref, w3_ref, w2_ref,
                      x_hbm, out_hbm, x_seg, acc_ref, wb1, wb3, wb2,
                      ld_sem, st_sem):
    s = pl.program_id(0)
    f = pl.program_id(1)

    off = soff_ref[s]
    nt = snt_ref[s]

    @pl.when(nt > 0)
    def _work():
        @pl.when(f == 0)
        def _load_seg():
            def stage(k, carry):
                cp = pltpu.make_async_copy(
                    x_hbm.at[pl.ds(pl.multiple_of(off + k * T, T), T), :],
                    x_seg.at[pl.ds(pl.multiple_of(k * T, T), T), :],
                    ld_sem)
                cp.start()
                cp.wait()
                return carry
            lax.fori_loop(0, nt, stage, 0)

        wb1[...] = w1_ref[0].astype(jnp.bfloat16)
        wb3[...] = w3_ref[0].astype(jnp.bfloat16)
        wb2[...] = w2_ref[0].astype(jnp.bfloat16)

        def tile_body(k, carry):
            rows = x_seg[pl.ds(pl.multiple_of(k * T, T), T), :]
            g = jnp.dot(rows, wb1[...], preferred_element_type=jnp.float32)
            u = jnp.dot(rows, wb3[...], preferred_element_type=jnp.float32)
            h = (g * jax.nn.sigmoid(g)) * u
            contrib = jnp.dot(h.astype(jnp.bfloat16), wb2[...],
                              preferred_element_type=jnp.float32)
            sl = pl.ds(pl.multiple_of(k * T, T), T)

            @pl.when(f == 0)
            def _init():
                acc_ref[sl, :] = contrib

            @pl.when(f > 0)
            def _accum():
                acc_ref[sl, :] = acc_ref[sl, :] + contrib

            return carry

        lax.fori_loop(0, nt, tile_body, 0)

        @pl.when(f == NF - 1)
        def _flush():
            def flush_tile(k, carry):
                cp = pltpu.make_async_copy(
                    acc_ref.at[pl.ds(pl.multiple_of(k * T, T), T), :],
                    out_hbm.at[pl.ds(pl.multiple_of(off + k * T, T), T), :],
                    st_sem)
                cp.start()
                cp.wait()
                return carry
            lax.fori_loop(0, nt, flush_tile, 0)


def _grouped_ffn(x_sorted, seg_expert, seg_nt, seg_off, w1, w3, w2):
    def wmap_in(s, f, se, snt, soff):
        return (se[s], 0, jnp.where(snt[s] > 0, f, 0))

    def wmap_out(s, f, se, snt, soff):
        return (se[s], jnp.where(snt[s] > 0, f, 0), 0)

    grid_spec = pltpu.PrefetchScalarGridSpec(
        num_scalar_prefetch=3,
        grid=(NSEG, NF),
        in_specs=[
            pl.BlockSpec((1, D, FB), wmap_in),
            pl.BlockSpec((1, D, FB), wmap_in),
            pl.BlockSpec((1, FB, D), wmap_out),
            pl.BlockSpec(memory_space=pl.ANY),
        ],
        out_specs=pl.BlockSpec(memory_space=pl.ANY),
        scratch_shapes=[
            pltpu.VMEM((SEG, D), jnp.bfloat16),
            pltpu.VMEM((SEG, D), jnp.float32),
            pltpu.VMEM((D, FB), jnp.bfloat16),
            pltpu.VMEM((D, FB), jnp.bfloat16),
            pltpu.VMEM((FB, D), jnp.bfloat16),
            pltpu.SemaphoreType.DMA,
            pltpu.SemaphoreType.DMA,
        ],
    )
    return pl.pallas_call(
        _grouped_ffn_body,
        grid_spec=grid_spec,
        out_shape=jax.ShapeDtypeStruct((P, D), jnp.float32),
    )(seg_expert, seg_nt, seg_off, w1, w3, w2, x_sorted)


def kernel(x, expert_idx, w1, w3, w2):
    b, s, d = x.shape
    x_flat = x.reshape(-1, d)
    idx = expert_idx.reshape(-1).astype(jnp.int32)
    n = idx.shape[0]

    oh = jax.nn.one_hot(idx, E, dtype=jnp.int32)
    incl = jnp.cumsum(oh, axis=0)
    counts = incl[-1]
    rank = jnp.sum((incl - oh) * oh, axis=1)
    padded = ((counts + T - 1) // T) * T
    offs = jnp.concatenate([jnp.zeros((1,), jnp.int32),
                            jnp.cumsum(padded)[:-1].astype(jnp.int32)])
    ntiles = (padded // T).astype(jnp.int32)
    dst = jnp.take(offs, idx) + rank
    pos_of_token = dst
    src_of_pos = jnp.zeros((P,), jnp.int32).at[dst].set(
        jnp.arange(n, dtype=jnp.int32))

    segs_e = (ntiles + TPS - 1) // TPS
    seg_start = jnp.concatenate([jnp.zeros((1,), jnp.int32),
                                 jnp.cumsum(segs_e)[:-1].astype(jnp.int32)])
    s_ids = jnp.arange(NSEG, dtype=jnp.int32)
    seg_expert = jnp.clip(
        jnp.sum((seg_start[None, :] <= s_ids[:, None]).astype(jnp.int32),
                axis=1) - 1, 0, E - 1).astype(jnp.int32)
    k_of_seg = s_ids - jnp.take(seg_start, seg_expert)
    seg_off = (jnp.take(offs, seg_expert) + k_of_seg * SEG).astype(jnp.int32)
    seg_nt = jnp.clip(jnp.take(ntiles, seg_expert) - k_of_seg * TPS,
                      0, TPS).astype(jnp.int32)

    x_sorted = _sc_gather_rows(x_flat, src_of_pos, P, 16).astype(jnp.bfloat16)

    y_sorted = _grouped_ffn(x_sorted, seg_expert, seg_nt, seg_off, w1, w3, w2)

    out = _sc_gather_rows(y_sorted, pos_of_token, n, 16)
    return out.reshape(b, s, d)
